# baseline (speedup 1.0000x reference)
import functools

import jax
import jax.numpy as jnp
from jax import lax
from jax.experimental import pallas as pl
from jax.experimental.pallas import tpu as pltpu
from jax.experimental.pallas import tpu_sc as plsc

VOCAB = 1000000
D = 64
VB = 65536
NBLK = -(-VOCAB // VB)
SUBS = 8
P = NBLK * VB

B = 4096
H = 200
NC = 2
NS = 16
NW = NC * NS
CPW = B // NW
K_CH = 8
NGRP = H // K_CH


def _proj_body(xt_ref, w_ref, b_ref, o_ref):
    xt = xt_ref[...]
    y = lax.dot_general(w_ref[...], xt, (((1,), (0,)), ((), ())),
                        preferred_element_type=jnp.float32)
    o_ref[...] = (y + b_ref[0]).reshape(VB)


def _gather_body(proj_hbm, idxt_hbm, outt_hbm, idx_v, val_v, sem):
    wid = lax.axis_index("s") * NC + lax.axis_index("c")
    c0 = wid * CPW
    pltpu.sync_copy(idxt_hbm.at[:, pl.ds(c0, CPW)], idx_v)

    def group(g, carry):
        base = g * K_CH
        copies = [pltpu.async_copy(
            proj_hbm.at[idx_v.at[base + k]],
            val_v.at[base + k], sem) for k in range(K_CH)]
        for c in copies:
            c.wait()
        return carry

    lax.fori_loop(0, NGRP, group, 0)
    pltpu.sync_copy(val_v, outt_hbm.at[:, pl.ds(c0, CPW)])


def kernel(feature, lan_emb, W_w, b_w, W_out, b_out):
    proj = pl.pallas_call(
        _proj_body,
        grid=(NBLK,),
        in_specs=[
            pl.BlockSpec((D, VB), lambda i: (0, i)),
            pl.BlockSpec((1, D), lambda i: (0, 0)),
            pl.BlockSpec(memory_space=pltpu.SMEM),
        ],
        out_specs=pl.BlockSpec((VB,), lambda i: (i,)),
        out_shape=jax.ShapeDtypeStruct((P,), jnp.float32),
    )(lan_emb.T, W_out, b_out)

    idxt = feature.astype(jnp.int32).T

    gather = functools.partial(
        pl.kernel,
        mesh=plsc.VectorSubcoreMesh(core_axis_name="c", subcore_axis_name="s"),
        out_type=jax.ShapeDtypeStruct((H, B), jnp.float32),
        scratch_types=[
            pltpu.VMEM((H, CPW), jnp.int32),
            pltpu.VMEM((H, CPW), jnp.float32),
            pltpu.SemaphoreType.DMA,
        ],
    )(_gather_body)
    outt = gather(proj, idxt)

    return outt.T


# speedup vs baseline: 1.0140x; 1.0140x over previous
import functools

import jax
import jax.numpy as jnp
from jax import lax
from jax.experimental import pallas as pl
from jax.experimental.pallas import tpu as pltpu
from jax.experimental.pallas import tpu_sc as plsc

VOCAB = 1000000
D = 64
VB = 32768
NBLK = -(-VOCAB // VB)
SUBS = 8
P = NBLK * VB

B = 4096
H = 200
NC = 2
NS = 16
NW = NC * NS
CPW = B // NW
DEPTH = 8


def _proj_body(xt_ref, w_ref, b_ref, o_ref):
    xt = xt_ref[...]
    y = lax.dot_general(w_ref[...], xt, (((1,), (0,)), ((), ())),
                        preferred_element_type=jnp.float32)
    o_ref[...] = (y + b_ref[0]).reshape(VB)


def _gather_body(proj_hbm, idxt_hbm, outt_hbm, idx_v, val_v, sem):
    wid = lax.axis_index("s") * NC + lax.axis_index("c")
    c0 = wid * CPW
    pltpu.sync_copy(idxt_hbm.at[:, pl.ds(c0, CPW)], idx_v)

    for k in range(DEPTH - 1):
        pltpu.async_copy(proj_hbm.at[idx_v.at[k]], val_v.at[k], sem)

    def chunk(g, carry):
        pltpu.async_copy(proj_hbm.at[idx_v.at[g]], val_v.at[g], sem)
        pltpu.make_async_copy(proj_hbm.at[idx_v.at[0]], val_v.at[0],
                              sem).wait()
        return carry

    lax.fori_loop(DEPTH - 1, H, chunk, 0)
    for k in range(DEPTH - 1):
        pltpu.make_async_copy(proj_hbm.at[idx_v.at[0]], val_v.at[0],
                              sem).wait()
    pltpu.sync_copy(val_v, outt_hbm.at[:, pl.ds(c0, CPW)])


def kernel(feature, lan_emb, W_w, b_w, W_out, b_out):
    proj = pl.pallas_call(
        _proj_body,
        grid=(NBLK,),
        in_specs=[
            pl.BlockSpec((D, VB), lambda i: (0, i)),
            pl.BlockSpec((1, D), lambda i: (0, 0)),
            pl.BlockSpec(memory_space=pltpu.SMEM),
        ],
        out_specs=pl.BlockSpec((VB,), lambda i: (i,)),
        out_shape=jax.ShapeDtypeStruct((P,), jnp.float32),
    )(lan_emb.T, W_out, b_out)

    idxt = feature.astype(jnp.int32).T

    gather = functools.partial(
        pl.kernel,
        mesh=plsc.VectorSubcoreMesh(core_axis_name="c", subcore_axis_name="s"),
        out_type=jax.ShapeDtypeStruct((H, B), jnp.float32),
        scratch_types=[
            pltpu.VMEM((H, CPW), jnp.int32),
            pltpu.VMEM((H, CPW), jnp.float32),
            pltpu.SemaphoreType.DMA,
        ],
    )(_gather_body)
    outt = gather(proj, idxt)

    return outt.T


# speedup vs baseline: 1.0231x; 1.0090x over previous
import functools

import jax
import jax.numpy as jnp
from jax import lax
from jax.experimental import pallas as pl
from jax.experimental.pallas import tpu as pltpu
from jax.experimental.pallas import tpu_sc as plsc

VOCAB = 1000000
D = 64
VB = 32768
NBLK = -(-VOCAB // VB)
SUBS = 8
P = NBLK * VB

B = 4096
H = 200
NC = 2
NS = 16
NW = NC * NS
NTOT = B * H
PW = NTOT // NW
CH = 1024
NCH = PW // CH
DEPTH = 4


def _proj_body(xt_ref, w_ref, b_ref, o_ref):
    xt = xt_ref[...]
    y = lax.dot_general(w_ref[...], xt, (((1,), (0,)), ((), ())),
                        preferred_element_type=jnp.float32)
    o_ref[...] = (y + b_ref[0]).reshape(VB)


def _gather_body(proj_hbm, idx_hbm, out_hbm, idx_v, val_v, sem):
    wid = lax.axis_index("s") * NC + lax.axis_index("c")
    p0 = wid * PW
    pltpu.sync_copy(idx_hbm.at[pl.ds(p0, PW)], idx_v)

    for k in range(DEPTH - 1):
        pltpu.async_copy(proj_hbm.at[idx_v.at[pl.ds(k * CH, CH)]],
                         val_v.at[pl.ds(k * CH, CH)], sem)

    def chunk(g, carry):
        pltpu.async_copy(proj_hbm.at[idx_v.at[pl.ds(g * CH, CH)]],
                         val_v.at[pl.ds(g * CH, CH)], sem)
        pltpu.make_async_copy(proj_hbm.at[idx_v.at[pl.ds(0, CH)]],
                              val_v.at[pl.ds(0, CH)], sem).wait()
        return carry

    lax.fori_loop(DEPTH - 1, NCH, chunk, 0)
    for k in range(DEPTH - 1):
        pltpu.make_async_copy(proj_hbm.at[idx_v.at[pl.ds(0, CH)]],
                              val_v.at[pl.ds(0, CH)], sem).wait()
    pltpu.sync_copy(val_v, out_hbm.at[pl.ds(p0, PW)])


def kernel(feature, lan_emb, W_w, b_w, W_out, b_out):
    proj = pl.pallas_call(
        _proj_body,
        grid=(NBLK,),
        in_specs=[
            pl.BlockSpec((D, VB), lambda i: (0, i)),
            pl.BlockSpec((1, D), lambda i: (0, 0)),
            pl.BlockSpec(memory_space=pltpu.SMEM),
        ],
        out_specs=pl.BlockSpec((VB,), lambda i: (i,)),
        out_shape=jax.ShapeDtypeStruct((P,), jnp.float32),
    )(lan_emb.T, W_out, b_out)

    idx_flat = feature.astype(jnp.int32).T.reshape(NTOT)

    gather = functools.partial(
        pl.kernel,
        mesh=plsc.VectorSubcoreMesh(core_axis_name="c", subcore_axis_name="s"),
        out_type=jax.ShapeDtypeStruct((NTOT,), jnp.float32),
        scratch_types=[
            pltpu.VMEM((PW,), jnp.int32),
            pltpu.VMEM((PW,), jnp.float32),
            pltpu.SemaphoreType.DMA,
        ],
    )(_gather_body)
    out_flat = gather(proj, idx_flat)

    return out_flat.reshape(H, B).T


# speedup vs baseline: 1.2128x; 1.1854x over previous
import functools

import jax
import jax.numpy as jnp
from jax import lax
from jax.experimental import pallas as pl
from jax.experimental.pallas import tpu as pltpu
from jax.experimental.pallas import tpu_sc as plsc

VOCAB = 1000000
D = 64
VB = 32768
NBLK = -(-VOCAB // VB)
SUBS = 8
P = NBLK * VB

B = 4096
H = 200
NC = 2
NS = 16
NW = NC * NS
NTOT = B * H
PW = NTOT // NW
CH = 1024
NCH = PW // CH
DEPTH = 4


def _proj_body(xt_ref, w_ref, b_ref, o_ref):
    xt = xt_ref[...]
    y = lax.dot_general(w_ref[...], xt, (((1,), (0,)), ((), ())),
                        preferred_element_type=jnp.float32)
    o_ref[...] = (y + b_ref[0]).reshape(VB)


def _gather_body(proj_hbm, idx_hbm, out_hbm, idx_v, val_v, proj_sh, sem):
    sid = lax.axis_index("s")
    wid = sid * NC + lax.axis_index("c")
    p0 = wid * PW
    pltpu.sync_copy(idx_hbm.at[pl.ds(p0, PW)], idx_v)

    @pl.when(sid == 0)
    def _stage():
        pltpu.sync_copy(proj_hbm, proj_sh)

    plsc.subcore_barrier()

    for k in range(DEPTH - 1):
        pltpu.async_copy(proj_sh.at[idx_v.at[pl.ds(k * CH, CH)]],
                         val_v.at[pl.ds(k * CH, CH)], sem)

    def chunk(g, carry):
        pltpu.async_copy(proj_sh.at[idx_v.at[pl.ds(g * CH, CH)]],
                         val_v.at[pl.ds(g * CH, CH)], sem)
        pltpu.make_async_copy(proj_sh.at[idx_v.at[pl.ds(0, CH)]],
                              val_v.at[pl.ds(0, CH)], sem).wait()
        return carry

    lax.fori_loop(DEPTH - 1, NCH, chunk, 0)
    for k in range(DEPTH - 1):
        pltpu.make_async_copy(proj_sh.at[idx_v.at[pl.ds(0, CH)]],
                              val_v.at[pl.ds(0, CH)], sem).wait()
    pltpu.sync_copy(val_v, out_hbm.at[pl.ds(p0, PW)])


def kernel(feature, lan_emb, W_w, b_w, W_out, b_out):
    proj = pl.pallas_call(
        _proj_body,
        grid=(NBLK,),
        in_specs=[
            pl.BlockSpec((D, VB), lambda i: (0, i)),
            pl.BlockSpec((1, D), lambda i: (0, 0)),
            pl.BlockSpec(memory_space=pltpu.SMEM),
        ],
        out_specs=pl.BlockSpec((VB,), lambda i: (i,)),
        out_shape=jax.ShapeDtypeStruct((P,), jnp.float32),
    )(lan_emb.T, W_out, b_out)

    idx_flat = feature.astype(jnp.int32).T.reshape(NTOT)

    gather = functools.partial(
        pl.kernel,
        mesh=plsc.VectorSubcoreMesh(core_axis_name="c", subcore_axis_name="s"),
        out_type=jax.ShapeDtypeStruct((NTOT,), jnp.float32),
        scratch_types=[
            pltpu.VMEM((PW,), jnp.int32),
            pltpu.VMEM((PW,), jnp.float32),
            pltpu.VMEM_SHARED((P,), jnp.float32),
            pltpu.SemaphoreType.DMA,
        ],
    )(_gather_body)
    out_flat = gather(proj, idx_flat)

    return out_flat.reshape(H, B).T


# speedup vs baseline: 1.2327x; 1.0164x over previous
import functools

import jax
import jax.numpy as jnp
from jax import lax
from jax.experimental import pallas as pl
from jax.experimental.pallas import tpu as pltpu
from jax.experimental.pallas import tpu_sc as plsc

VOCAB = 1000000
D = 64
VB = 32768
NBLK = -(-VOCAB // VB)
P = NBLK * VB

B = 4096
H = 200
NC = 2
NS = 16
NW = NC * NS
NTOT = B * H
PW = NTOT // NW
CH = 1024
NCH = PW // CH
DEPTH = 4


def _proj_body(xt_ref, w_ref, b_ref, o_ref):
    xt = xt_ref[...]
    y = lax.dot_general(w_ref[...], xt, (((1,), (0,)), ((), ())),
                        preferred_element_type=jnp.float32)
    o_ref[...] = (y + b_ref[0]).reshape(VB)


def _gather_body(proj_hbm, idx_hbm, out_hbm, idx_v, val_v, proj_sh, sem):
    sid = lax.axis_index("s")
    wid = sid * NC + lax.axis_index("c")
    p0 = wid * PW
    pltpu.sync_copy(idx_hbm.at[pl.ds(p0, PW)], idx_v)

    @pl.when(sid == 0)
    def _stage():
        pltpu.sync_copy(proj_hbm, proj_sh)

    plsc.subcore_barrier()

    for k in range(DEPTH - 1):
        pltpu.async_copy(proj_sh.at[idx_v.at[pl.ds(k * CH, CH)]],
                         val_v.at[pl.ds(k * CH, CH)], sem)

    def chunk(g, carry):
        pltpu.async_copy(proj_sh.at[idx_v.at[pl.ds(g * CH, CH)]],
                         val_v.at[pl.ds(g * CH, CH)], sem)
        pltpu.make_async_copy(proj_sh.at[idx_v.at[pl.ds(0, CH)]],
                              val_v.at[pl.ds(0, CH)], sem).wait()
        return carry

    lax.fori_loop(DEPTH - 1, NCH, chunk, 0)
    for k in range(DEPTH - 1):
        pltpu.make_async_copy(proj_sh.at[idx_v.at[pl.ds(0, CH)]],
                              val_v.at[pl.ds(0, CH)], sem).wait()
    pltpu.sync_copy(val_v, out_hbm.at[pl.ds(p0, PW)])


def kernel(feature, lan_emb, W_w, b_w, W_out, b_out):
    proj = pl.pallas_call(
        _proj_body,
        grid=(NBLK,),
        in_specs=[
            pl.BlockSpec((D, VB), lambda i: (0, i)),
            pl.BlockSpec((1, D), lambda i: (0, 0)),
            pl.BlockSpec(memory_space=pltpu.SMEM),
        ],
        out_specs=pl.BlockSpec((VB,), lambda i: (i,)),
        out_shape=jax.ShapeDtypeStruct((P,), jnp.float32),
    )(lan_emb.T, W_out, b_out)

    idx_flat = feature.astype(jnp.int32).T.reshape(NTOT)

    gather = functools.partial(
        pl.kernel,
        mesh=plsc.VectorSubcoreMesh(core_axis_name="c", subcore_axis_name="s"),
        out_type=jax.ShapeDtypeStruct((NTOT,), jnp.float32),
        scratch_types=[
            pltpu.VMEM((PW,), jnp.int32),
            pltpu.VMEM((PW,), jnp.float32),
            pltpu.VMEM_SHARED((P,), jnp.float32),
            pltpu.SemaphoreType.DMA,
        ],
    )(_gather_body)
    out_flat = gather(proj, idx_flat)

    return out_flat.reshape(H, B).T
